# Initial kernel scaffold; baseline (speedup 1.0000x reference)
import jax
import jax.numpy as jnp
from jax.experimental import pallas as pl


def kernel(x, edge_index, Win, bin_, Wr, br, We1, be1, We2, be2, Wk1, bk1, Wk2, bk2, Wout, bout):
    raise NotImplementedError("write your pallas kernel here")


# speedup vs baseline: 1.0899x; 1.0899x over previous
import functools

import jax
import jax.numpy as jnp
from jax.experimental import pallas as pl
from jax.experimental.pallas import tpu as pltpu

N_NODES = 10000
IN_DIM = 256
HID = 256
N_EXP = 8
WEAK = 64
NB = 400


def _in_proj_body(x_ref, w_ref, b_ref, o_ref):
    o_ref[...] = (
        jnp.dot(x_ref[...], w_ref[...], preferred_element_type=jnp.float32)
        + b_ref[...]
    )


def _matmul_bias(x, w, b):
    n, d = x.shape
    dout = w.shape[1]
    grid = n // NB
    return pl.pallas_call(
        _in_proj_body,
        grid=(grid,),
        in_specs=[
            pl.BlockSpec((NB, d), lambda i: (i, 0)),
            pl.BlockSpec((d, dout), lambda i: (0, 0)),
            pl.BlockSpec((1, dout), lambda i: (0, 0)),
        ],
        out_specs=pl.BlockSpec((NB, dout), lambda i: (i, 0)),
        out_shape=jax.ShapeDtypeStruct((n, dout), jnp.float32),
    )(x, w, b.reshape(1, dout))


def _layer_body(
    h_ref, agg_ref, wr_ref, br_ref, we1_ref, be1_ref, we2_ref, be2_ref,
    wk1_ref, bk1_ref, wk2_ref, bk2_ref, o_ref,
):
    h = h_ref[...]
    agg = agg_ref[...]
    wr = wr_ref[...]
    logits = (
        jnp.dot(h, wr[:HID], preferred_element_type=jnp.float32)
        + jnp.dot(agg, wr[HID:], preferred_element_type=jnp.float32)
        + br_ref[...]
    )

    iota = jax.lax.broadcasted_iota(jnp.int32, (NB, N_EXP), 1)
    m1 = jnp.max(logits, axis=1, keepdims=True)
    i1 = jnp.min(jnp.where(logits == m1, iota, N_EXP), axis=1, keepdims=True)
    oh1 = iota == i1
    masked = jnp.where(oh1, -3e38, logits)
    m2 = jnp.max(masked, axis=1, keepdims=True)
    i2 = jnp.min(jnp.where(masked == m2, iota, N_EXP), axis=1, keepdims=True)
    oh2 = iota == i2
    t = jnp.exp(m2 - m1)
    g1 = 1.0 / (1.0 + t)
    g2 = 1.0 - g1
    comb = g1 * oh1.astype(jnp.float32) + g2 * oh2.astype(jnp.float32)
    conf = 1.0 / jnp.sum(jnp.exp(logits - m1), axis=1, keepdims=True)

    weak = (
        jnp.dot(
            jnp.maximum(
                jnp.dot(h, wk1_ref[...], preferred_element_type=jnp.float32)
                + bk1_ref[...],
                0.0,
            ),
            wk2_ref[...],
            preferred_element_type=jnp.float32,
        )
        + bk2_ref[...]
    )

    moe = weak
    for e in range(N_EXP):
        hid = jnp.maximum(
            jnp.dot(h, we1_ref[e], preferred_element_type=jnp.float32)
            + be1_ref[e],
            0.0,
        )
        eo = jnp.dot(hid, we2_ref[e], preferred_element_type=jnp.float32) + be2_ref[e]
        moe = moe + comb[:, e : e + 1] * eo

    o_ref[...] = h + conf * moe


def _layer_call(h, agg, wr, br, we1, be1, we2, be2, wk1, bk1, wk2, bk2):
    n = h.shape[0]
    grid = n // NB
    full = lambda *s: pl.BlockSpec(s, lambda i: (0,) * len(s))
    return pl.pallas_call(
        _layer_body,
        grid=(grid,),
        in_specs=[
            pl.BlockSpec((NB, HID), lambda i: (i, 0)),
            pl.BlockSpec((NB, HID), lambda i: (i, 0)),
            full(2 * HID, N_EXP),
            full(1, N_EXP),
            full(N_EXP, HID, HID),
            full(N_EXP, 1, HID),
            full(N_EXP, HID, HID),
            full(N_EXP, 1, HID),
            full(HID, WEAK),
            full(1, WEAK),
            full(WEAK, HID),
            full(1, HID),
        ],
        out_specs=pl.BlockSpec((NB, HID), lambda i: (i, 0)),
        out_shape=jax.ShapeDtypeStruct((n, HID), jnp.float32),
    )(
        h, agg, wr, br.reshape(1, N_EXP),
        we1, be1.reshape(N_EXP, 1, HID), we2, be2.reshape(N_EXP, 1, HID),
        wk1, bk1.reshape(1, WEAK), wk2, bk2.reshape(1, HID),
    )


def kernel(x, edge_index, Win, bin_, Wr, br, We1, be1, We2, be2, Wk1, bk1, Wk2, bk2, Wout, bout):
    n = x.shape[0]
    src = edge_index[0]
    dst = edge_index[1]
    h = _matmul_bias(x, Win, bin_)
    n_layers = Wr.shape[0]
    for l in range(n_layers):
        msg = jnp.take(h, src, axis=0)
        agg = jax.ops.segment_sum(msg, dst, num_segments=n)
        deg = jax.ops.segment_sum(jnp.ones((src.shape[0], 1), h.dtype), dst, num_segments=n)
        agg = agg / jnp.maximum(deg, 1.0)
        h = _layer_call(
            h, agg, Wr[l], br[l], We1[l], be1[l], We2[l], be2[l],
            Wk1[l], bk1[l], Wk2[l], bk2[l],
        )
    return _matmul_bias(h, Wout, bout)


# speedup vs baseline: 10.9339x; 10.0318x over previous
import functools

import jax
import jax.numpy as jnp
from jax import lax
from jax.experimental import pallas as pl
from jax.experimental.pallas import tpu as pltpu
from jax.experimental.pallas import tpu_sc as plsc

N_NODES = 10000
HID = 256
N_EXP = 8
WEAK = 64
NB = 400

QW = 16
NTILES = 32
CHUNK = 128
CHUNKS = 40
EPAD = NTILES * CHUNKS * CHUNK
NPAD = 10240
ROWS_PER_TILE = NPAD // 16


def _sc_agg_body(q_hbm, src_hbm, dst_hbm, zero_hbm, out_hbm,
                 sidx_v, didx_v, rows_v, acc_sh, sem):
    c = lax.axis_index("c")
    s = lax.axis_index("s")
    w = c * 16 + s
    r0 = s * ROWS_PER_TILE
    pltpu.sync_copy(zero_hbm.at[pl.ds(r0, ROWS_PER_TILE)],
                    acc_sh.at[pl.ds(r0, ROWS_PER_TILE)])
    pltpu.sync_copy(src_hbm.at[pl.ds(w * CHUNKS, CHUNKS)], sidx_v)
    pltpu.sync_copy(dst_hbm.at[pl.ds(w * CHUNKS, CHUNKS)], didx_v)
    plsc.subcore_barrier()

    def chunk(j, carry):
        pltpu.async_copy(q_hbm.at[sidx_v.at[j]], rows_v, sem).wait()
        pltpu.sync_copy(rows_v, acc_sh.at[didx_v.at[j]], add=True)
        return carry

    lax.fori_loop(0, CHUNKS, chunk, 0)
    plsc.subcore_barrier()
    pltpu.sync_copy(acc_sh.at[pl.ds(r0, ROWS_PER_TILE)],
                    out_hbm.at[c, pl.ds(r0, ROWS_PER_TILE)])


@functools.partial(jax.jit, static_argnums=())
def _sc_agg(q, src2d, dst2d, zeros):
    mesh = plsc.VectorSubcoreMesh(core_axis_name="c", subcore_axis_name="s")
    f = pl.kernel(
        _sc_agg_body,
        mesh=mesh,
        compiler_params=pltpu.CompilerParams(use_tc_tiling_on_sc=False),
        out_type=jax.ShapeDtypeStruct((2, NPAD, QW), jnp.float32),
        scratch_types=[
            pltpu.VMEM((CHUNKS, CHUNK), jnp.int32),
            pltpu.VMEM((CHUNKS, CHUNK), jnp.int32),
            pltpu.VMEM((CHUNK, QW), jnp.float32),
            pltpu.VMEM_SHARED((NPAD, QW), jnp.float32),
            pltpu.SemaphoreType.DMA,
        ],
    )
    return f(q, src2d, dst2d, zeros)


def _in_proj_body(x_ref, w_ref, b_ref, wq_ref, h_ref, q_ref):
    h = jnp.dot(x_ref[...], w_ref[...], preferred_element_type=jnp.float32) + b_ref[...]
    h_ref[...] = h
    ones_col = (jax.lax.broadcasted_iota(jnp.int32, (NB, QW), 1) == N_EXP
                ).astype(jnp.float32)
    q_ref[...] = jnp.dot(h, wq_ref[...], preferred_element_type=jnp.float32) + ones_col


def _in_proj(x, w, b, wq):
    n, d = x.shape
    dout = w.shape[1]
    return pl.pallas_call(
        _in_proj_body,
        grid=(n // NB,),
        in_specs=[
            pl.BlockSpec((NB, d), lambda i: (i, 0)),
            pl.BlockSpec((d, dout), lambda i: (0, 0)),
            pl.BlockSpec((1, dout), lambda i: (0, 0)),
            pl.BlockSpec((dout, QW), lambda i: (0, 0)),
        ],
        out_specs=[
            pl.BlockSpec((NB, dout), lambda i: (i, 0)),
            pl.BlockSpec((NB, QW), lambda i: (i, 0)),
        ],
        out_shape=[
            jax.ShapeDtypeStruct((n, dout), jnp.float32),
            jax.ShapeDtypeStruct((n, QW), jnp.float32),
        ],
    )(x, w, b.reshape(1, dout), wq)


def _moe_update(h, s0, s1, wr1_ref, br_ref, we1_ref, be1_ref, we2_ref, be2_ref,
                wk1_ref, bk1_ref, wk2_ref, bk2_ref):
    s = s0 + s1
    deg = jnp.maximum(s[:, N_EXP:N_EXP + 1], 1.0)
    logits = (
        jnp.dot(h, wr1_ref[...], preferred_element_type=jnp.float32)
        + s[:, :N_EXP] / deg
        + br_ref[...]
    )

    iota = jax.lax.broadcasted_iota(jnp.int32, (NB, N_EXP), 1)
    m1 = jnp.max(logits, axis=1, keepdims=True)
    i1 = jnp.min(jnp.where(logits == m1, iota, N_EXP), axis=1, keepdims=True)
    oh1 = iota == i1
    masked = jnp.where(oh1, -3e38, logits)
    m2 = jnp.max(masked, axis=1, keepdims=True)
    i2 = jnp.min(jnp.where(masked == m2, iota, N_EXP), axis=1, keepdims=True)
    oh2 = iota == i2
    t = jnp.exp(m2 - m1)
    g1 = 1.0 / (1.0 + t)
    g2 = 1.0 - g1
    comb = g1 * oh1.astype(jnp.float32) + g2 * oh2.astype(jnp.float32)
    conf = 1.0 / jnp.sum(jnp.exp(logits - m1), axis=1, keepdims=True)

    weak = (
        jnp.dot(
            jnp.maximum(
                jnp.dot(h, wk1_ref[...], preferred_element_type=jnp.float32)
                + bk1_ref[...],
                0.0,
            ),
            wk2_ref[...],
            preferred_element_type=jnp.float32,
        )
        + bk2_ref[...]
    )

    moe = weak
    for e in range(N_EXP):
        hid = jnp.maximum(
            jnp.dot(h, we1_ref[e], preferred_element_type=jnp.float32) + be1_ref[e],
            0.0,
        )
        eo = jnp.dot(hid, we2_ref[e], preferred_element_type=jnp.float32) + be2_ref[e]
        moe = moe + comb[:, e:e + 1] * eo

    return h + conf * moe


def _layer_mid_body(h_ref, s0_ref, s1_ref, wr1_ref, br_ref, we1_ref, be1_ref,
                    we2_ref, be2_ref, wk1_ref, bk1_ref, wk2_ref, bk2_ref,
                    wq_ref, h_out_ref, q_ref):
    hn = _moe_update(h_ref[...], s0_ref[...], s1_ref[...], wr1_ref, br_ref,
                     we1_ref, be1_ref, we2_ref, be2_ref,
                     wk1_ref, bk1_ref, wk2_ref, bk2_ref)
    h_out_ref[...] = hn
    ones_col = (jax.lax.broadcasted_iota(jnp.int32, (NB, QW), 1) == N_EXP
                ).astype(jnp.float32)
    q_ref[...] = jnp.dot(hn, wq_ref[...], preferred_element_type=jnp.float32) + ones_col


def _layer_last_body(h_ref, s0_ref, s1_ref, wr1_ref, br_ref, we1_ref, be1_ref,
                     we2_ref, be2_ref, wk1_ref, bk1_ref, wk2_ref, bk2_ref,
                     wo_ref, bo_ref, o_ref):
    hn = _moe_update(h_ref[...], s0_ref[...], s1_ref[...], wr1_ref, br_ref,
                     we1_ref, be1_ref, we2_ref, be2_ref,
                     wk1_ref, bk1_ref, wk2_ref, bk2_ref)
    o_ref[...] = jnp.dot(hn, wo_ref[...], preferred_element_type=jnp.float32) + bo_ref[...]


def _layer_specs(extra):
    full = lambda *s: pl.BlockSpec(s, lambda i, _ns=len(s): (0,) * _ns)
    return [
        pl.BlockSpec((NB, HID), lambda i: (i, 0)),
        pl.BlockSpec((NB, QW), lambda i: (i, 0)),
        pl.BlockSpec((NB, QW), lambda i: (i, 0)),
        full(HID, N_EXP),
        full(1, N_EXP),
        full(N_EXP, HID, HID),
        full(N_EXP, 1, HID),
        full(N_EXP, HID, HID),
        full(N_EXP, 1, HID),
        full(HID, WEAK),
        full(1, WEAK),
        full(WEAK, HID),
        full(1, HID),
    ] + [full(*s) for s in extra]


def _layer_args(h, s_pair, wr, br, we1, be1, we2, be2, wk1, bk1, wk2, bk2):
    return (
        h, s_pair[0], s_pair[1], wr[:HID], br.reshape(1, N_EXP),
        we1, be1.reshape(N_EXP, 1, HID), we2, be2.reshape(N_EXP, 1, HID),
        wk1, bk1.reshape(1, WEAK), wk2, bk2.reshape(1, HID),
    )


def kernel(x, edge_index, Win, bin_, Wr, br, We1, be1, We2, be2, Wk1, bk1, Wk2, bk2, Wout, bout):
    n = x.shape[0]
    src = edge_index[0]
    dst = edge_index[1]

    npad_e = EPAD - src.shape[0]
    pad_ar = jnp.arange(npad_e, dtype=jnp.int32)
    src2d = jnp.concatenate([src, pad_ar % n]).reshape(EPAD // CHUNK, CHUNK)
    dst2d = jnp.concatenate([dst, n + pad_ar % (NPAD - n)]).reshape(EPAD // CHUNK, CHUNK)
    zeros = jnp.zeros((NPAD, QW), jnp.float32)

    wq = [jnp.pad(Wr[l][HID:], ((0, 0), (0, QW - N_EXP))) for l in range(Wr.shape[0])]

    h, q = _in_proj(x, Win, bin_, wq[0])

    s = _sc_agg(q, src2d, dst2d, zeros)
    s_pair = (s[0, :n], s[1, :n])

    h, q = pl.pallas_call(
        _layer_mid_body,
        grid=(n // NB,),
        in_specs=_layer_specs([(HID, QW)]),
        out_specs=[
            pl.BlockSpec((NB, HID), lambda i: (i, 0)),
            pl.BlockSpec((NB, QW), lambda i: (i, 0)),
        ],
        out_shape=[
            jax.ShapeDtypeStruct((n, HID), jnp.float32),
            jax.ShapeDtypeStruct((n, QW), jnp.float32),
        ],
    )(*_layer_args(h, s_pair, Wr[0], br[0], We1[0], be1[0], We2[0], be2[0],
                   Wk1[0], bk1[0], Wk2[0], bk2[0]), wq[1])

    s = _sc_agg(q, src2d, dst2d, zeros)
    s_pair = (s[0, :n], s[1, :n])

    out = pl.pallas_call(
        _layer_last_body,
        grid=(n // NB,),
        in_specs=_layer_specs([(HID, HID), (1, HID)]),
        out_specs=pl.BlockSpec((NB, HID), lambda i: (i, 0)),
        out_shape=jax.ShapeDtypeStruct((n, HID), jnp.float32),
    )(*_layer_args(h, s_pair, Wr[1], br[1], We1[1], be1[1], We2[1], be2[1],
                   Wk1[1], bk1[1], Wk2[1], bk2[1]), Wout, bout.reshape(1, HID))

    return out


# speedup vs baseline: 12.9825x; 1.1874x over previous
import functools

import jax
import jax.numpy as jnp
from jax import lax
from jax.experimental import pallas as pl
from jax.experimental.pallas import tpu as pltpu
from jax.experimental.pallas import tpu_sc as plsc

N_NODES = 10000
HID = 256
N_EXP = 8
WEAK = 64
NB = 400

QW = 16
NTILES = 32
CHUNK = 128
CHUNKS = 40
EPAD = NTILES * CHUNKS * CHUNK
NPAD = 10240
ROWS_PER_TILE = NPAD // 16


def _sc_agg_body(q_hbm, src_hbm, dst_hbm, zero_hbm, out_hbm,
                 sidx_v, didx_v, rows_v, acc_sh, sem):
    c = lax.axis_index("c")
    s = lax.axis_index("s")
    w = c * 16 + s
    r0 = s * ROWS_PER_TILE
    pltpu.sync_copy(zero_hbm.at[pl.ds(r0, ROWS_PER_TILE)],
                    acc_sh.at[pl.ds(r0, ROWS_PER_TILE)])
    pltpu.sync_copy(src_hbm.at[pl.ds(w * CHUNKS, CHUNKS)], sidx_v)
    pltpu.sync_copy(dst_hbm.at[pl.ds(w * CHUNKS, CHUNKS)], didx_v)
    plsc.subcore_barrier()

    def fire(j, carry):
        pltpu.async_copy(q_hbm.at[sidx_v.at[j]],
                         rows_v.at[pl.ds(j * CHUNK, CHUNK)], sem)
        return carry

    lax.fori_loop(0, CHUNKS, fire, 0)
    pltpu.make_async_copy(q_hbm.at[pl.ds(0, CHUNKS * CHUNK)], rows_v, sem).wait()

    def scat(j, carry):
        pltpu.sync_copy(rows_v.at[pl.ds(j * CHUNK, CHUNK)],
                        acc_sh.at[didx_v.at[j]], add=True)
        return carry

    lax.fori_loop(0, CHUNKS, scat, 0)
    plsc.subcore_barrier()
    pltpu.sync_copy(acc_sh.at[pl.ds(r0, ROWS_PER_TILE)],
                    out_hbm.at[c, pl.ds(r0, ROWS_PER_TILE)])


@functools.partial(jax.jit, static_argnums=())
def _sc_agg(q, src2d, dst2d, zeros):
    mesh = plsc.VectorSubcoreMesh(core_axis_name="c", subcore_axis_name="s")
    f = pl.kernel(
        _sc_agg_body,
        mesh=mesh,
        compiler_params=pltpu.CompilerParams(use_tc_tiling_on_sc=False),
        out_type=jax.ShapeDtypeStruct((2, NPAD, QW), jnp.float32),
        scratch_types=[
            pltpu.VMEM((CHUNKS, CHUNK), jnp.int32),
            pltpu.VMEM((CHUNKS, CHUNK), jnp.int32),
            pltpu.VMEM((CHUNKS * CHUNK, QW), jnp.float32),
            pltpu.VMEM_SHARED((NPAD, QW), jnp.float32),
            pltpu.SemaphoreType.DMA,
        ],
    )
    return f(q, src2d, dst2d, zeros)


def _in_proj_body(x_ref, w_ref, b_ref, wq_ref, h_ref, q_ref):
    h = jnp.dot(x_ref[...], w_ref[...], preferred_element_type=jnp.float32) + b_ref[...]
    h_ref[...] = h
    ones_col = (jax.lax.broadcasted_iota(jnp.int32, (NB, QW), 1) == N_EXP
                ).astype(jnp.float32)
    q_ref[...] = jnp.dot(h, wq_ref[...], preferred_element_type=jnp.float32) + ones_col


def _in_proj(x, w, b, wq):
    n, d = x.shape
    dout = w.shape[1]
    return pl.pallas_call(
        _in_proj_body,
        grid=(n // NB,),
        in_specs=[
            pl.BlockSpec((NB, d), lambda i: (i, 0)),
            pl.BlockSpec((d, dout), lambda i: (0, 0)),
            pl.BlockSpec((1, dout), lambda i: (0, 0)),
            pl.BlockSpec((dout, QW), lambda i: (0, 0)),
        ],
        out_specs=[
            pl.BlockSpec((NB, dout), lambda i: (i, 0)),
            pl.BlockSpec((NB, QW), lambda i: (i, 0)),
        ],
        out_shape=[
            jax.ShapeDtypeStruct((n, dout), jnp.float32),
            jax.ShapeDtypeStruct((n, QW), jnp.float32),
        ],
    )(x, w, b.reshape(1, dout), wq)


def _moe_update(h, s0, s1, wr1_ref, br_ref, we1_ref, be1_ref, we2_ref, be2_ref,
                wk1_ref, bk1_ref, wk2_ref, bk2_ref):
    s = s0 + s1
    deg = jnp.maximum(s[:, N_EXP:N_EXP + 1], 1.0)
    logits = (
        jnp.dot(h, wr1_ref[...], preferred_element_type=jnp.float32)
        + s[:, :N_EXP] / deg
        + br_ref[...]
    )

    iota = jax.lax.broadcasted_iota(jnp.int32, (NB, N_EXP), 1)
    m1 = jnp.max(logits, axis=1, keepdims=True)
    i1 = jnp.min(jnp.where(logits == m1, iota, N_EXP), axis=1, keepdims=True)
    oh1 = iota == i1
    masked = jnp.where(oh1, -3e38, logits)
    m2 = jnp.max(masked, axis=1, keepdims=True)
    i2 = jnp.min(jnp.where(masked == m2, iota, N_EXP), axis=1, keepdims=True)
    oh2 = iota == i2
    t = jnp.exp(m2 - m1)
    g1 = 1.0 / (1.0 + t)
    g2 = 1.0 - g1
    comb = g1 * oh1.astype(jnp.float32) + g2 * oh2.astype(jnp.float32)
    conf = 1.0 / jnp.sum(jnp.exp(logits - m1), axis=1, keepdims=True)

    weak = (
        jnp.dot(
            jnp.maximum(
                jnp.dot(h, wk1_ref[...], preferred_element_type=jnp.float32)
                + bk1_ref[...],
                0.0,
            ),
            wk2_ref[...],
            preferred_element_type=jnp.float32,
        )
        + bk2_ref[...]
    )

    moe = weak
    for e in range(N_EXP):
        hid = jnp.maximum(
            jnp.dot(h, we1_ref[e], preferred_element_type=jnp.float32,
                    precision=jax.lax.Precision.DEFAULT) + be1_ref[e],
            0.0,
        )
        eo = jnp.dot(hid, we2_ref[e], preferred_element_type=jnp.float32,
                     precision=jax.lax.Precision.DEFAULT) + be2_ref[e]
        moe = moe + comb[:, e:e + 1] * eo

    return h + conf * moe


def _layer_mid_body(h_ref, s0_ref, s1_ref, wr1_ref, br_ref, we1_ref, be1_ref,
                    we2_ref, be2_ref, wk1_ref, bk1_ref, wk2_ref, bk2_ref,
                    wq_ref, h_out_ref, q_ref):
    hn = _moe_update(h_ref[...], s0_ref[...], s1_ref[...], wr1_ref, br_ref,
                     we1_ref, be1_ref, we2_ref, be2_ref,
                     wk1_ref, bk1_ref, wk2_ref, bk2_ref)
    h_out_ref[...] = hn
    ones_col = (jax.lax.broadcasted_iota(jnp.int32, (NB, QW), 1) == N_EXP
                ).astype(jnp.float32)
    q_ref[...] = jnp.dot(hn, wq_ref[...], preferred_element_type=jnp.float32) + ones_col


def _layer_last_body(h_ref, s0_ref, s1_ref, wr1_ref, br_ref, we1_ref, be1_ref,
                     we2_ref, be2_ref, wk1_ref, bk1_ref, wk2_ref, bk2_ref,
                     wo_ref, bo_ref, o_ref):
    hn = _moe_update(h_ref[...], s0_ref[...], s1_ref[...], wr1_ref, br_ref,
                     we1_ref, be1_ref, we2_ref, be2_ref,
                     wk1_ref, bk1_ref, wk2_ref, bk2_ref)
    o_ref[...] = jnp.dot(hn, wo_ref[...], preferred_element_type=jnp.float32) + bo_ref[...]


def _layer_specs(extra):
    full = lambda *s: pl.BlockSpec(s, lambda i, _ns=len(s): (0,) * _ns)
    return [
        pl.BlockSpec((NB, HID), lambda i: (i, 0)),
        pl.BlockSpec((NB, QW), lambda i: (i, 0)),
        pl.BlockSpec((NB, QW), lambda i: (i, 0)),
        full(HID, N_EXP),
        full(1, N_EXP),
        full(N_EXP, HID, HID),
        full(N_EXP, 1, HID),
        full(N_EXP, HID, HID),
        full(N_EXP, 1, HID),
        full(HID, WEAK),
        full(1, WEAK),
        full(WEAK, HID),
        full(1, HID),
    ] + [full(*s) for s in extra]


def _layer_args(h, s_pair, wr, br, we1, be1, we2, be2, wk1, bk1, wk2, bk2):
    return (
        h, s_pair[0], s_pair[1], wr[:HID], br.reshape(1, N_EXP),
        we1, be1.reshape(N_EXP, 1, HID), we2, be2.reshape(N_EXP, 1, HID),
        wk1, bk1.reshape(1, WEAK), wk2, bk2.reshape(1, HID),
    )


def kernel(x, edge_index, Win, bin_, Wr, br, We1, be1, We2, be2, Wk1, bk1, Wk2, bk2, Wout, bout):
    n = x.shape[0]
    src = edge_index[0]
    dst = edge_index[1]

    npad_e = EPAD - src.shape[0]
    pad_ar = jnp.arange(npad_e, dtype=jnp.int32)
    src2d = jnp.concatenate([src, pad_ar % n]).reshape(EPAD // CHUNK, CHUNK)
    dst2d = jnp.concatenate([dst, n + pad_ar % (NPAD - n)]).reshape(EPAD // CHUNK, CHUNK)
    zeros = jnp.zeros((NPAD, QW), jnp.float32)

    wq = [jnp.pad(Wr[l][HID:], ((0, 0), (0, QW - N_EXP))) for l in range(Wr.shape[0])]

    h, q = _in_proj(x, Win, bin_, wq[0])

    s = _sc_agg(q, src2d, dst2d, zeros)
    s_pair = (s[0, :n], s[1, :n])

    h, q = pl.pallas_call(
        _layer_mid_body,
        grid=(n // NB,),
        in_specs=_layer_specs([(HID, QW)]),
        out_specs=[
            pl.BlockSpec((NB, HID), lambda i: (i, 0)),
            pl.BlockSpec((NB, QW), lambda i: (i, 0)),
        ],
        out_shape=[
            jax.ShapeDtypeStruct((n, HID), jnp.float32),
            jax.ShapeDtypeStruct((n, QW), jnp.float32),
        ],
    )(*_layer_args(h, s_pair, Wr[0], br[0], We1[0], be1[0], We2[0], be2[0],
                   Wk1[0], bk1[0], Wk2[0], bk2[0]), wq[1])

    s = _sc_agg(q, src2d, dst2d, zeros)
    s_pair = (s[0, :n], s[1, :n])

    out = pl.pallas_call(
        _layer_last_body,
        grid=(n // NB,),
        in_specs=_layer_specs([(HID, HID), (1, HID)]),
        out_specs=pl.BlockSpec((NB, HID), lambda i: (i, 0)),
        out_shape=jax.ShapeDtypeStruct((n, HID), jnp.float32),
    )(*_layer_args(h, s_pair, Wr[1], br[1], We1[1], be1[1], We2[1], be2[1],
                   Wk1[1], bk1[1], Wk2[1], bk2[1]), Wout, bout.reshape(1, HID))

    return out


# speedup vs baseline: 13.8807x; 1.0692x over previous
import functools

import jax
import jax.numpy as jnp
from jax import lax
from jax.experimental import pallas as pl
from jax.experimental.pallas import tpu as pltpu
from jax.experimental.pallas import tpu_sc as plsc

N_NODES = 10000
HID = 256
N_EXP = 8
WEAK = 64
NB = 400

QW = 16
NTILES = 32
CHUNK = 128
CHUNKS = 40
EPAD = NTILES * CHUNKS * CHUNK
NPAD = 10400
ROWS_PER_TILE = NPAD // 16


def _sc_agg_body(q_hbm, src_hbm, dst_hbm, zero_hbm, out_hbm,
                 sidx_v, didx_v, rows_v, acc_sh, sem):
    c = lax.axis_index("c")
    s = lax.axis_index("s")
    w = c * 16 + s
    r0 = s * ROWS_PER_TILE
    pltpu.sync_copy(zero_hbm.at[pl.ds(r0, ROWS_PER_TILE)],
                    acc_sh.at[pl.ds(r0, ROWS_PER_TILE)])
    pltpu.sync_copy(src_hbm.at[pl.ds(w * CHUNKS, CHUNKS)], sidx_v)
    pltpu.sync_copy(dst_hbm.at[pl.ds(w * CHUNKS, CHUNKS)], didx_v)
    plsc.subcore_barrier()

    def fire(j, carry):
        pltpu.async_copy(q_hbm.at[sidx_v.at[j]],
                         rows_v.at[pl.ds(j * CHUNK, CHUNK)], sem)
        return carry

    lax.fori_loop(0, CHUNKS, fire, 0)
    pltpu.make_async_copy(q_hbm.at[pl.ds(0, CHUNKS * CHUNK)], rows_v, sem).wait()

    def scat(j, carry):
        pltpu.sync_copy(rows_v.at[pl.ds(j * CHUNK, CHUNK)],
                        acc_sh.at[didx_v.at[j]], add=True)
        return carry

    lax.fori_loop(0, CHUNKS, scat, 0)
    plsc.subcore_barrier()
    pltpu.sync_copy(acc_sh.at[pl.ds(r0, ROWS_PER_TILE)],
                    out_hbm.at[c, pl.ds(r0, ROWS_PER_TILE)])


@functools.partial(jax.jit, static_argnums=())
def _sc_agg(q, src2d, dst2d, zeros):
    mesh = plsc.VectorSubcoreMesh(core_axis_name="c", subcore_axis_name="s")
    f = pl.kernel(
        _sc_agg_body,
        mesh=mesh,
        compiler_params=pltpu.CompilerParams(use_tc_tiling_on_sc=False),
        out_type=jax.ShapeDtypeStruct((2, NPAD, QW), jnp.float32),
        scratch_types=[
            pltpu.VMEM((CHUNKS, CHUNK), jnp.int32),
            pltpu.VMEM((CHUNKS, CHUNK), jnp.int32),
            pltpu.VMEM((CHUNKS * CHUNK, QW), jnp.float32),
            pltpu.VMEM_SHARED((NPAD, QW), jnp.float32),
            pltpu.SemaphoreType.DMA,
        ],
    )
    return f(q, src2d, dst2d, zeros)


def _q0_body(x_ref, wq_ref, c0_ref, q_ref):
    q_ref[...] = (
        jnp.dot(x_ref[...], wq_ref[...], preferred_element_type=jnp.float32)
        + c0_ref[...]
    )


def _q0_proj(x, wq_eff, c0):
    n, d = x.shape
    return pl.pallas_call(
        _q0_body,
        grid=(n // NB,),
        in_specs=[
            pl.BlockSpec((NB, d), lambda i: (i, 0)),
            pl.BlockSpec((d, QW), lambda i: (0, 0)),
            pl.BlockSpec((1, QW), lambda i: (0, 0)),
        ],
        out_specs=pl.BlockSpec((NB, QW), lambda i: (i, 0)),
        out_shape=jax.ShapeDtypeStruct((n, QW), jnp.float32),
    )(x, wq_eff, c0)


def _moe_update(h, s_ref, wr1_ref, br_ref, we1_ref, be1_ref, we2_ref, be2_ref,
                wk1_ref, bk1_ref, wk2_ref, bk2_ref):
    s = s_ref[0] + s_ref[1]
    deg = jnp.maximum(s[:, N_EXP:N_EXP + 1], 1.0)
    logits = (
        jnp.dot(h, wr1_ref[...], preferred_element_type=jnp.float32)
        + s[:, :N_EXP] / deg
        + br_ref[...]
    )

    iota = jax.lax.broadcasted_iota(jnp.int32, (NB, N_EXP), 1)
    m1 = jnp.max(logits, axis=1, keepdims=True)
    i1 = jnp.min(jnp.where(logits == m1, iota, N_EXP), axis=1, keepdims=True)
    oh1 = iota == i1
    masked = jnp.where(oh1, -3e38, logits)
    m2 = jnp.max(masked, axis=1, keepdims=True)
    i2 = jnp.min(jnp.where(masked == m2, iota, N_EXP), axis=1, keepdims=True)
    oh2 = iota == i2
    t = jnp.exp(m2 - m1)
    g1 = 1.0 / (1.0 + t)
    g2 = 1.0 - g1
    comb = g1 * oh1.astype(jnp.float32) + g2 * oh2.astype(jnp.float32)
    conf = 1.0 / jnp.sum(jnp.exp(logits - m1), axis=1, keepdims=True)

    weak = (
        jnp.dot(
            jnp.maximum(
                jnp.dot(h, wk1_ref[...], preferred_element_type=jnp.float32)
                + bk1_ref[...],
                0.0,
            ),
            wk2_ref[...],
            preferred_element_type=jnp.float32,
        )
        + bk2_ref[...]
    )

    moe = weak
    for e in range(N_EXP):
        hid = jnp.maximum(
            jnp.dot(h, we1_ref[e], preferred_element_type=jnp.float32,
                    precision=jax.lax.Precision.DEFAULT) + be1_ref[e],
            0.0,
        )
        eo = jnp.dot(hid, we2_ref[e], preferred_element_type=jnp.float32,
                     precision=jax.lax.Precision.DEFAULT) + be2_ref[e]
        moe = moe + comb[:, e:e + 1] * eo

    return h + conf * moe


def _layer0_body(x_ref, s_ref, win_ref, binp_ref, wr1_ref, br_ref, we1_ref,
                 be1_ref, we2_ref, be2_ref, wk1_ref, bk1_ref, wk2_ref, bk2_ref,
                 wq_ref, h_out_ref, q_ref):
    h = (jnp.dot(x_ref[...], win_ref[...], preferred_element_type=jnp.float32)
         + binp_ref[...])
    hn = _moe_update(h, s_ref, wr1_ref, br_ref, we1_ref, be1_ref, we2_ref,
                     be2_ref, wk1_ref, bk1_ref, wk2_ref, bk2_ref)
    h_out_ref[...] = hn
    ones_col = (jax.lax.broadcasted_iota(jnp.int32, (NB, QW), 1) == N_EXP
                ).astype(jnp.float32)
    q_ref[...] = jnp.dot(hn, wq_ref[...], preferred_element_type=jnp.float32) + ones_col


def _layer_last_body(h_ref, s_ref, wr1_ref, br_ref, we1_ref, be1_ref,
                     we2_ref, be2_ref, wk1_ref, bk1_ref, wk2_ref, bk2_ref,
                     wo_ref, bo_ref, o_ref):
    hn = _moe_update(h_ref[...], s_ref, wr1_ref, br_ref,
                     we1_ref, be1_ref, we2_ref, be2_ref,
                     wk1_ref, bk1_ref, wk2_ref, bk2_ref)
    o_ref[...] = jnp.dot(hn, wo_ref[...], preferred_element_type=jnp.float32) + bo_ref[...]


def _layer_specs(pre, extra):
    full = lambda *s: pl.BlockSpec(s, lambda i, _ns=len(s): (0,) * _ns)
    return [
        pl.BlockSpec((NB, HID), lambda i: (i, 0)),
        pl.BlockSpec((2, NB, QW), lambda i: (0, i, 0)),
    ] + [full(*s) for s in pre] + [full(*s) for s in (
        (HID, N_EXP),
        (1, N_EXP),
        (N_EXP, HID, HID),
        (N_EXP, 1, HID),
        (N_EXP, HID, HID),
        (N_EXP, 1, HID),
        (HID, WEAK),
        (1, WEAK),
        (WEAK, HID),
        (1, HID),
    )] + [full(*s) for s in extra]


def _layer_args(wr, br, we1, be1, we2, be2, wk1, bk1, wk2, bk2):
    return (
        wr[:HID], br.reshape(1, N_EXP),
        we1, be1.reshape(N_EXP, 1, HID), we2, be2.reshape(N_EXP, 1, HID),
        wk1, bk1.reshape(1, WEAK), wk2, bk2.reshape(1, HID),
    )


def kernel(x, edge_index, Win, bin_, Wr, br, We1, be1, We2, be2, Wk1, bk1, Wk2, bk2, Wout, bout):
    n = x.shape[0]
    src = edge_index[0]
    dst = edge_index[1]

    npad_e = EPAD - src.shape[0]
    pad_ar = jnp.arange(npad_e, dtype=jnp.int32)
    src2d = jnp.concatenate([src, pad_ar % n]).reshape(EPAD // CHUNK, CHUNK)
    dst2d = jnp.concatenate([dst, n + pad_ar % (NPAD - n)]).reshape(EPAD // CHUNK, CHUNK)
    zeros = jnp.zeros((NPAD, QW), jnp.float32)

    wq = [jnp.pad(Wr[l][HID:], ((0, 0), (0, QW - N_EXP))) for l in range(Wr.shape[0])]
    wq0_eff = Win @ wq[0]
    c0 = (bin_ @ wq[0] + (jnp.arange(QW) == N_EXP)).reshape(1, QW).astype(jnp.float32)

    q = _q0_proj(x, wq0_eff, c0)
    s = _sc_agg(q, src2d, dst2d, zeros)

    h, q = pl.pallas_call(
        _layer0_body,
        grid=(n // NB,),
        in_specs=_layer_specs([(HID, HID), (1, HID)], [(HID, QW)]),
        out_specs=[
            pl.BlockSpec((NB, HID), lambda i: (i, 0)),
            pl.BlockSpec((NB, QW), lambda i: (i, 0)),
        ],
        out_shape=[
            jax.ShapeDtypeStruct((n, HID), jnp.float32),
            jax.ShapeDtypeStruct((n, QW), jnp.float32),
        ],
    )(x, s, Win, bin_.reshape(1, HID),
      *_layer_args(Wr[0], br[0], We1[0], be1[0], We2[0], be2[0],
                   Wk1[0], bk1[0], Wk2[0], bk2[0]), wq[1])

    s = _sc_agg(q, src2d, dst2d, zeros)

    out = pl.pallas_call(
        _layer_last_body,
        grid=(n // NB,),
        in_specs=_layer_specs([], [(HID, HID), (1, HID)]),
        out_specs=pl.BlockSpec((NB, HID), lambda i: (i, 0)),
        out_shape=jax.ShapeDtypeStruct((n, HID), jnp.float32),
    )(h, s, *_layer_args(Wr[1], br[1], We1[1], be1[1], We2[1], be2[1],
                         Wk1[1], bk1[1], Wk2[1], bk2[1]), Wout, bout.reshape(1, HID))

    return out


# speedup vs baseline: 15.5156x; 1.1178x over previous
import functools

import jax
import jax.numpy as jnp
from jax import lax
from jax.experimental import pallas as pl
from jax.experimental.pallas import tpu as pltpu
from jax.experimental.pallas import tpu_sc as plsc

N_NODES = 10000
HID = 256
N_EXP = 8
WEAK = 64
NB = 400

QW = 16
NTILES = 32
CHUNK = 128
CHUNKS = 40
EPAD = NTILES * CHUNKS * CHUNK
NPAD = 10400
ROWS_PER_TILE = NPAD // 16


def _sc_agg_body(q_hbm, edges_hbm, zero_hbm, out_hbm,
                 sidx_v, didx_v, rows_v, acc_sh, sem):
    c = lax.axis_index("c")
    s = lax.axis_index("s")
    w = c * 16 + s
    r0 = s * ROWS_PER_TILE
    pltpu.sync_copy(zero_hbm.at[pl.ds(r0, ROWS_PER_TILE)],
                    acc_sh.at[pl.ds(r0, ROWS_PER_TILE)])
    pltpu.sync_copy(edges_hbm.at[0, pl.ds(w * CHUNKS, CHUNKS)], sidx_v)
    pltpu.sync_copy(edges_hbm.at[1, pl.ds(w * CHUNKS, CHUNKS)], didx_v)
    plsc.subcore_barrier()

    def fire(j, carry):
        pltpu.async_copy(q_hbm.at[sidx_v.at[j]],
                         rows_v.at[pl.ds(j * CHUNK, CHUNK)], sem)
        return carry

    lax.fori_loop(0, CHUNKS, fire, 0)
    pltpu.make_async_copy(q_hbm.at[pl.ds(0, CHUNKS * CHUNK)], rows_v, sem).wait()

    def scat(j, carry):
        pltpu.sync_copy(rows_v.at[pl.ds(j * CHUNK, CHUNK)],
                        acc_sh.at[didx_v.at[j]], add=True)
        return carry

    lax.fori_loop(0, CHUNKS, scat, 0)
    plsc.subcore_barrier()
    pltpu.sync_copy(acc_sh.at[pl.ds(r0, ROWS_PER_TILE)],
                    out_hbm.at[c, pl.ds(r0, ROWS_PER_TILE)])


@functools.partial(jax.jit, static_argnums=())
def _sc_agg(q, edges2d, zeros):
    mesh = plsc.VectorSubcoreMesh(core_axis_name="c", subcore_axis_name="s")
    f = pl.kernel(
        _sc_agg_body,
        mesh=mesh,
        compiler_params=pltpu.CompilerParams(use_tc_tiling_on_sc=False),
        out_type=jax.ShapeDtypeStruct((2, NPAD, QW), jnp.float32),
        scratch_types=[
            pltpu.VMEM((CHUNKS, CHUNK), jnp.int32),
            pltpu.VMEM((CHUNKS, CHUNK), jnp.int32),
            pltpu.VMEM((CHUNKS * CHUNK, QW), jnp.float32),
            pltpu.VMEM_SHARED((NPAD, QW), jnp.float32),
            pltpu.SemaphoreType.DMA,
        ],
    )
    return f(q, edges2d, zeros)


NBQ = 2000


def _q0_body(x_ref, wq_ref, c0_ref, q_ref):
    q_ref[...] = (
        jnp.dot(x_ref[...], wq_ref[...], preferred_element_type=jnp.float32)
        + c0_ref[...]
    )


def _q0_proj(x, wq_eff, c0):
    n, d = x.shape
    return pl.pallas_call(
        _q0_body,
        grid=(n // NBQ,),
        in_specs=[
            pl.BlockSpec((NBQ, d), lambda i: (i, 0)),
            pl.BlockSpec((d, 128), lambda i: (0, 0)),
            pl.BlockSpec((1, 128), lambda i: (0, 0)),
        ],
        out_specs=pl.BlockSpec((NBQ, 128), lambda i: (i, 0)),
        out_shape=jax.ShapeDtypeStruct((n, 128), jnp.float32),
    )(x, wq_eff, c0)


def _moe_update(h, s_ref, wr1_ref, br_ref, we1_ref, be1_ref, we2_ref, be2_ref,
                wk1_ref, bk1_ref, wk2_ref, bk2_ref):
    s = s_ref[0] + s_ref[1]
    deg = jnp.maximum(s[:, N_EXP:N_EXP + 1], 1.0)
    logits = (
        jnp.dot(h, wr1_ref[...], preferred_element_type=jnp.float32)
        + s[:, :N_EXP] / deg
        + br_ref[...]
    )

    iota = jax.lax.broadcasted_iota(jnp.int32, (NB, N_EXP), 1)
    m1 = jnp.max(logits, axis=1, keepdims=True)
    i1 = jnp.min(jnp.where(logits == m1, iota, N_EXP), axis=1, keepdims=True)
    oh1 = iota == i1
    masked = jnp.where(oh1, -3e38, logits)
    m2 = jnp.max(masked, axis=1, keepdims=True)
    i2 = jnp.min(jnp.where(masked == m2, iota, N_EXP), axis=1, keepdims=True)
    oh2 = iota == i2
    t = jnp.exp(m2 - m1)
    g1 = 1.0 / (1.0 + t)
    g2 = 1.0 - g1
    comb = g1 * oh1.astype(jnp.float32) + g2 * oh2.astype(jnp.float32)
    conf = 1.0 / jnp.sum(jnp.exp(logits - m1), axis=1, keepdims=True)

    weak = (
        jnp.dot(
            jnp.maximum(
                jnp.dot(h, wk1_ref[...], preferred_element_type=jnp.float32)
                + bk1_ref[...],
                0.0,
            ),
            wk2_ref[...],
            preferred_element_type=jnp.float32,
        )
        + bk2_ref[...]
    )

    moe = weak
    for e in range(N_EXP):
        hid = jnp.maximum(
            jnp.dot(h, we1_ref[e], preferred_element_type=jnp.float32,
                    precision=jax.lax.Precision.DEFAULT) + be1_ref[e],
            0.0,
        )
        eo = jnp.dot(hid, we2_ref[e], preferred_element_type=jnp.float32,
                     precision=jax.lax.Precision.DEFAULT) + be2_ref[e]
        moe = moe + comb[:, e:e + 1] * eo

    return h + conf * moe


def _layer0_body(x_ref, s_ref, win_ref, binp_ref, wr1_ref, br_ref, we1_ref,
                 be1_ref, we2_ref, be2_ref, wk1_ref, bk1_ref, wk2_ref, bk2_ref,
                 wq_ref, h_out_ref, q_ref):
    h = (jnp.dot(x_ref[...], win_ref[...], preferred_element_type=jnp.float32)
         + binp_ref[...])
    hn = _moe_update(h, s_ref, wr1_ref, br_ref, we1_ref, be1_ref, we2_ref,
                     be2_ref, wk1_ref, bk1_ref, wk2_ref, bk2_ref)
    h_out_ref[...] = hn
    ones_col = (jax.lax.broadcasted_iota(jnp.int32, (NB, 128), 1) == N_EXP
                ).astype(jnp.float32)
    q_ref[...] = jnp.dot(hn, wq_ref[...], preferred_element_type=jnp.float32) + ones_col


def _layer_last_body(h_ref, s_ref, wr1_ref, br_ref, we1_ref, be1_ref,
                     we2_ref, be2_ref, wk1_ref, bk1_ref, wk2_ref, bk2_ref,
                     wo_ref, bo_ref, o_ref):
    hn = _moe_update(h_ref[...], s_ref, wr1_ref, br_ref,
                     we1_ref, be1_ref, we2_ref, be2_ref,
                     wk1_ref, bk1_ref, wk2_ref, bk2_ref)
    o_ref[...] = jnp.dot(hn, wo_ref[...], preferred_element_type=jnp.float32) + bo_ref[...]


def _layer_specs(pre, extra):
    full = lambda *s: pl.BlockSpec(s, lambda i, _ns=len(s): (0,) * _ns)
    return [
        pl.BlockSpec((NB, HID), lambda i: (i, 0)),
        pl.BlockSpec((2, NB, QW), lambda i: (0, i, 0)),
    ] + [full(*s) for s in pre] + [full(*s) for s in (
        (HID, N_EXP),
        (1, N_EXP),
        (N_EXP, HID, HID),
        (N_EXP, 1, HID),
        (N_EXP, HID, HID),
        (N_EXP, 1, HID),
        (HID, WEAK),
        (1, WEAK),
        (WEAK, HID),
        (1, HID),
    )] + [full(*s) for s in extra]


def _layer_args(wr, br, we1, be1, we2, be2, wk1, bk1, wk2, bk2):
    return (
        wr[:HID], br.reshape(1, N_EXP),
        we1, be1.reshape(N_EXP, 1, HID), we2, be2.reshape(N_EXP, 1, HID),
        wk1, bk1.reshape(1, WEAK), wk2, bk2.reshape(1, HID),
    )


def kernel(x, edge_index, Win, bin_, Wr, br, We1, be1, We2, be2, Wk1, bk1, Wk2, bk2, Wout, bout):
    n = x.shape[0]
    src = edge_index[0]
    dst = edge_index[1]

    npad_e = EPAD - src.shape[0]
    pad_ar = jnp.arange(npad_e, dtype=jnp.int32)
    edges2d = jnp.stack([
        jnp.concatenate([src * 8, (pad_ar % n) * 8]).reshape(EPAD // CHUNK, CHUNK),
        jnp.concatenate([dst, n + pad_ar % (NPAD - n)]).reshape(EPAD // CHUNK, CHUNK),
    ])
    zeros = jnp.zeros((NPAD, QW), jnp.float32)

    wq = [jnp.pad(Wr[l][HID:], ((0, 0), (0, 128 - N_EXP))) for l in range(Wr.shape[0])]
    wq0_eff = Win @ wq[0]
    c0 = (bin_ @ wq[0] + (jnp.arange(128) == N_EXP)).reshape(1, 128).astype(jnp.float32)

    q = _q0_proj(x, wq0_eff, c0)
    s = _sc_agg(q.reshape(n * 8, QW), edges2d, zeros)

    h, q = pl.pallas_call(
        _layer0_body,
        grid=(n // NB,),
        in_specs=_layer_specs([(HID, HID), (1, HID)], [(HID, 128)]),
        out_specs=[
            pl.BlockSpec((NB, HID), lambda i: (i, 0)),
            pl.BlockSpec((NB, 128), lambda i: (i, 0)),
        ],
        out_shape=[
            jax.ShapeDtypeStruct((n, HID), jnp.float32),
            jax.ShapeDtypeStruct((n, 128), jnp.float32),
        ],
    )(x, s, Win, bin_.reshape(1, HID),
      *_layer_args(Wr[0], br[0], We1[0], be1[0], We2[0], be2[0],
                   Wk1[0], bk1[0], Wk2[0], bk2[0]), wq[1])

    s = _sc_agg(q.reshape(n * 8, QW), edges2d, zeros)

    out = pl.pallas_call(
        _layer_last_body,
        grid=(n // NB,),
        in_specs=_layer_specs([], [(HID, HID), (1, HID)]),
        out_specs=pl.BlockSpec((NB, HID), lambda i: (i, 0)),
        out_shape=jax.ShapeDtypeStruct((n, HID), jnp.float32),
    )(h, s, *_layer_args(Wr[1], br[1], We1[1], be1[1], We2[1], be2[1],
                         Wk1[1], bk1[1], Wk2[1], bk2[1]), Wout, bout.reshape(1, HID))

    return out


# speedup vs baseline: 16.4784x; 1.0621x over previous
import functools

import jax
import jax.numpy as jnp
from jax import lax
from jax.experimental import pallas as pl
from jax.experimental.pallas import tpu as pltpu
from jax.experimental.pallas import tpu_sc as plsc

N_NODES = 10000
HID = 256
N_EXP = 8
WEAK = 64
NB = 1000

QW = 16
NTILES = 32
CHUNK = 128
CHUNKS = 40
EPAD = NTILES * CHUNKS * CHUNK
NPAD = 12000
ROWS_PER_TILE = NPAD // 16


def _sc_agg_body(q_hbm, edges_hbm, zero_hbm, out_hbm,
                 sidx_v, didx_v, rows_v, acc_sh, sem):
    c = lax.axis_index("c")
    s = lax.axis_index("s")
    w = c * 16 + s
    r0 = s * ROWS_PER_TILE
    pltpu.sync_copy(zero_hbm.at[pl.ds(r0, ROWS_PER_TILE)],
                    acc_sh.at[pl.ds(r0, ROWS_PER_TILE)])
    pltpu.sync_copy(edges_hbm.at[0, pl.ds(w * CHUNKS, CHUNKS)], sidx_v)
    pltpu.sync_copy(edges_hbm.at[1, pl.ds(w * CHUNKS, CHUNKS)], didx_v)
    plsc.subcore_barrier()

    def fire(j, carry):
        pltpu.async_copy(q_hbm.at[sidx_v.at[j]],
                         rows_v.at[pl.ds(j * CHUNK, CHUNK)], sem)
        return carry

    lax.fori_loop(0, CHUNKS, fire, 0)
    pltpu.make_async_copy(q_hbm.at[pl.ds(0, CHUNKS * CHUNK)], rows_v, sem).wait()

    def scat(j, carry):
        pltpu.sync_copy(rows_v.at[pl.ds(j * CHUNK, CHUNK)],
                        acc_sh.at[didx_v.at[j]], add=True)
        return carry

    lax.fori_loop(0, CHUNKS, scat, 0)
    plsc.subcore_barrier()
    pltpu.sync_copy(acc_sh.at[pl.ds(r0, ROWS_PER_TILE)],
                    out_hbm.at[c, pl.ds(r0, ROWS_PER_TILE)])


@functools.partial(jax.jit, static_argnums=())
def _sc_agg(q, edges2d, zeros):
    mesh = plsc.VectorSubcoreMesh(core_axis_name="c", subcore_axis_name="s")
    f = pl.kernel(
        _sc_agg_body,
        mesh=mesh,
        compiler_params=pltpu.CompilerParams(use_tc_tiling_on_sc=False),
        out_type=jax.ShapeDtypeStruct((2, NPAD, QW), jnp.float32),
        scratch_types=[
            pltpu.VMEM((CHUNKS, CHUNK), jnp.int32),
            pltpu.VMEM((CHUNKS, CHUNK), jnp.int32),
            pltpu.VMEM((CHUNKS * CHUNK, QW), jnp.float32),
            pltpu.VMEM_SHARED((NPAD, QW), jnp.float32),
            pltpu.SemaphoreType.DMA,
        ],
    )
    return f(q, edges2d, zeros)


NBQ = 2000


def _q0_body(x_ref, wq_ref, c0_ref, q_ref):
    q_ref[...] = (
        jnp.dot(x_ref[...], wq_ref[...], preferred_element_type=jnp.float32)
        + c0_ref[...]
    )


def _q0_proj(x, wq_eff, c0):
    n, d = x.shape
    return pl.pallas_call(
        _q0_body,
        grid=(n // NBQ,),
        in_specs=[
            pl.BlockSpec((NBQ, d), lambda i: (i, 0)),
            pl.BlockSpec((d, 128), lambda i: (0, 0)),
            pl.BlockSpec((1, 128), lambda i: (0, 0)),
        ],
        out_specs=pl.BlockSpec((NBQ, 128), lambda i: (i, 0)),
        out_shape=jax.ShapeDtypeStruct((n, 128), jnp.float32),
    )(x, wq_eff, c0)


def _moe_update(h, s_ref, wr1_ref, br_ref, we1_ref, be1_ref, we2_ref, be2_ref,
                wk1_ref, bk1_ref, wk2_ref, bk2_ref):
    s = s_ref[0] + s_ref[1]
    deg = jnp.maximum(s[:, N_EXP:N_EXP + 1], 1.0)
    logits = (
        jnp.dot(h, wr1_ref[...], preferred_element_type=jnp.float32)
        + s[:, :N_EXP] / deg
        + br_ref[...]
    )

    iota = jax.lax.broadcasted_iota(jnp.int32, (NB, N_EXP), 1)
    m1 = jnp.max(logits, axis=1, keepdims=True)
    i1 = jnp.min(jnp.where(logits == m1, iota, N_EXP), axis=1, keepdims=True)
    oh1 = iota == i1
    masked = jnp.where(oh1, -3e38, logits)
    m2 = jnp.max(masked, axis=1, keepdims=True)
    i2 = jnp.min(jnp.where(masked == m2, iota, N_EXP), axis=1, keepdims=True)
    oh2 = iota == i2
    t = jnp.exp(m2 - m1)
    g1 = 1.0 / (1.0 + t)
    g2 = 1.0 - g1
    comb = g1 * oh1.astype(jnp.float32) + g2 * oh2.astype(jnp.float32)
    conf = 1.0 / jnp.sum(jnp.exp(logits - m1), axis=1, keepdims=True)

    weak = (
        jnp.dot(
            jnp.maximum(
                jnp.dot(h, wk1_ref[...], preferred_element_type=jnp.float32)
                + bk1_ref[...],
                0.0,
            ),
            wk2_ref[...],
            preferred_element_type=jnp.float32,
        )
        + bk2_ref[...]
    )

    moe = weak
    h16 = h.astype(jnp.bfloat16)
    for e in range(N_EXP):
        hid = jnp.maximum(
            jnp.dot(h16, we1_ref[e].astype(jnp.bfloat16),
                    preferred_element_type=jnp.float32) + be1_ref[e],
            0.0,
        )
        eo = jnp.dot(hid.astype(jnp.bfloat16), we2_ref[e].astype(jnp.bfloat16),
                     preferred_element_type=jnp.float32) + be2_ref[e]
        moe = moe + comb[:, e:e + 1] * eo

    return h + conf * moe


def _layer0_body(x_ref, s_ref, win_ref, binp_ref, wr1_ref, br_ref, we1_ref,
                 be1_ref, we2_ref, be2_ref, wk1_ref, bk1_ref, wk2_ref, bk2_ref,
                 wq_ref, h_out_ref, q_ref):
    h = (jnp.dot(x_ref[...], win_ref[...], preferred_element_type=jnp.float32)
         + binp_ref[...])
    hn = _moe_update(h, s_ref, wr1_ref, br_ref, we1_ref, be1_ref, we2_ref,
                     be2_ref, wk1_ref, bk1_ref, wk2_ref, bk2_ref)
    h_out_ref[...] = hn
    ones_col = (jax.lax.broadcasted_iota(jnp.int32, (NB, 128), 1) == N_EXP
                ).astype(jnp.float32)
    q_ref[...] = jnp.dot(hn, wq_ref[...], preferred_element_type=jnp.float32) + ones_col


def _layer_last_body(h_ref, s_ref, wr1_ref, br_ref, we1_ref, be1_ref,
                     we2_ref, be2_ref, wk1_ref, bk1_ref, wk2_ref, bk2_ref,
                     wo_ref, bo_ref, o_ref):
    hn = _moe_update(h_ref[...], s_ref, wr1_ref, br_ref,
                     we1_ref, be1_ref, we2_ref, be2_ref,
                     wk1_ref, bk1_ref, wk2_ref, bk2_ref)
    o_ref[...] = jnp.dot(hn, wo_ref[...], preferred_element_type=jnp.float32) + bo_ref[...]


def _layer_specs(pre, extra):
    full = lambda *s: pl.BlockSpec(s, lambda i, _ns=len(s): (0,) * _ns)
    return [
        pl.BlockSpec((NB, HID), lambda i: (i, 0)),
        pl.BlockSpec((2, NB, QW), lambda i: (0, i, 0)),
    ] + [full(*s) for s in pre] + [full(*s) for s in (
        (HID, N_EXP),
        (1, N_EXP),
        (N_EXP, HID, HID),
        (N_EXP, 1, HID),
        (N_EXP, HID, HID),
        (N_EXP, 1, HID),
        (HID, WEAK),
        (1, WEAK),
        (WEAK, HID),
        (1, HID),
    )] + [full(*s) for s in extra]


def _layer_args(wr, br, we1, be1, we2, be2, wk1, bk1, wk2, bk2):
    return (
        wr[:HID], br.reshape(1, N_EXP),
        we1, be1.reshape(N_EXP, 1, HID), we2, be2.reshape(N_EXP, 1, HID),
        wk1, bk1.reshape(1, WEAK), wk2, bk2.reshape(1, HID),
    )


def kernel(x, edge_index, Win, bin_, Wr, br, We1, be1, We2, be2, Wk1, bk1, Wk2, bk2, Wout, bout):
    n = x.shape[0]
    src = edge_index[0]
    dst = edge_index[1]

    npad_e = EPAD - src.shape[0]
    pad_ar = jnp.arange(npad_e, dtype=jnp.int32)
    edges2d = jnp.stack([
        jnp.concatenate([src * 8, (pad_ar % n) * 8]).reshape(EPAD // CHUNK, CHUNK),
        jnp.concatenate([dst, n + pad_ar % (NPAD - n)]).reshape(EPAD // CHUNK, CHUNK),
    ])
    zeros = jnp.zeros((NPAD, QW), jnp.float32)

    wq = [jnp.pad(Wr[l][HID:], ((0, 0), (0, 128 - N_EXP))) for l in range(Wr.shape[0])]
    wq0_eff = Win @ wq[0]
    c0 = (bin_ @ wq[0] + (jnp.arange(128) == N_EXP)).reshape(1, 128).astype(jnp.float32)

    q = _q0_proj(x, wq0_eff, c0)
    s = _sc_agg(q.reshape(n * 8, QW), edges2d, zeros)

    h, q = pl.pallas_call(
        _layer0_body,
        grid=(n // NB,),
        in_specs=_layer_specs([(HID, HID), (1, HID)], [(HID, 128)]),
        out_specs=[
            pl.BlockSpec((NB, HID), lambda i: (i, 0)),
            pl.BlockSpec((NB, 128), lambda i: (i, 0)),
        ],
        out_shape=[
            jax.ShapeDtypeStruct((n, HID), jnp.float32),
            jax.ShapeDtypeStruct((n, 128), jnp.float32),
        ],
    )(x, s, Win, bin_.reshape(1, HID),
      *_layer_args(Wr[0], br[0], We1[0], be1[0], We2[0], be2[0],
                   Wk1[0], bk1[0], Wk2[0], bk2[0]), wq[1])

    s = _sc_agg(q.reshape(n * 8, QW), edges2d, zeros)

    out = pl.pallas_call(
        _layer_last_body,
        grid=(n // NB,),
        in_specs=_layer_specs([], [(HID, HID), (1, HID)]),
        out_specs=pl.BlockSpec((NB, HID), lambda i: (i, 0)),
        out_shape=jax.ShapeDtypeStruct((n, HID), jnp.float32),
    )(h, s, *_layer_args(Wr[1], br[1], We1[1], be1[1], We2[1], be2[1],
                         Wk1[1], bk1[1], Wk2[1], bk2[1]), Wout, bout.reshape(1, HID))

    return out


# speedup vs baseline: 16.6237x; 1.0088x over previous
import functools

import jax
import jax.numpy as jnp
from jax import lax
from jax.experimental import pallas as pl
from jax.experimental.pallas import tpu as pltpu
from jax.experimental.pallas import tpu_sc as plsc

N_NODES = 10000
HID = 256
N_EXP = 8
WEAK = 64
NB = 2000

QW = 16
NTILES = 32
CHUNK = 128
CHUNKS = 40
EPAD = NTILES * CHUNKS * CHUNK
NPAD = 12000
ROWS_PER_TILE = NPAD // 16


def _sc_agg_body(q_hbm, edges_hbm, zero_hbm, out_hbm,
                 sidx_v, didx_v, rows_v, acc_sh, sem):
    c = lax.axis_index("c")
    s = lax.axis_index("s")
    w = c * 16 + s
    r0 = s * ROWS_PER_TILE
    pltpu.sync_copy(zero_hbm.at[pl.ds(r0, ROWS_PER_TILE)],
                    acc_sh.at[pl.ds(r0, ROWS_PER_TILE)])
    pltpu.sync_copy(edges_hbm.at[0, pl.ds(w * CHUNKS, CHUNKS)], sidx_v)
    pltpu.sync_copy(edges_hbm.at[1, pl.ds(w * CHUNKS, CHUNKS)], didx_v)
    plsc.subcore_barrier()

    def fire(j, carry):
        pltpu.async_copy(q_hbm.at[sidx_v.at[j]],
                         rows_v.at[pl.ds(j * CHUNK, CHUNK)], sem)
        return carry

    lax.fori_loop(0, CHUNKS, fire, 0)
    pltpu.make_async_copy(q_hbm.at[pl.ds(0, CHUNKS * CHUNK)], rows_v, sem).wait()

    def scat(j, carry):
        pltpu.sync_copy(rows_v.at[pl.ds(j * CHUNK, CHUNK)],
                        acc_sh.at[didx_v.at[j]], add=True)
        return carry

    lax.fori_loop(0, CHUNKS, scat, 0)
    plsc.subcore_barrier()
    pltpu.sync_copy(acc_sh.at[pl.ds(r0, ROWS_PER_TILE)],
                    out_hbm.at[c, pl.ds(r0, ROWS_PER_TILE)])


@functools.partial(jax.jit, static_argnums=())
def _sc_agg(q, edges2d, zeros):
    mesh = plsc.VectorSubcoreMesh(core_axis_name="c", subcore_axis_name="s")
    f = pl.kernel(
        _sc_agg_body,
        mesh=mesh,
        compiler_params=pltpu.CompilerParams(use_tc_tiling_on_sc=False),
        out_type=jax.ShapeDtypeStruct((2, NPAD, QW), jnp.float32),
        scratch_types=[
            pltpu.VMEM((CHUNKS, CHUNK), jnp.int32),
            pltpu.VMEM((CHUNKS, CHUNK), jnp.int32),
            pltpu.VMEM((CHUNKS * CHUNK, QW), jnp.float32),
            pltpu.VMEM_SHARED((NPAD, QW), jnp.float32),
            pltpu.SemaphoreType.DMA,
        ],
    )
    return f(q, edges2d, zeros)


NBQ = 2000


def _q0_body(x_ref, wq_ref, c0_ref, q_ref):
    q_ref[...] = (
        jnp.dot(x_ref[...], wq_ref[...], preferred_element_type=jnp.float32)
        + c0_ref[...]
    )


def _q0_proj(x, wq_eff, c0):
    n, d = x.shape
    return pl.pallas_call(
        _q0_body,
        grid=(n // NBQ,),
        in_specs=[
            pl.BlockSpec((NBQ, d), lambda i: (i, 0)),
            pl.BlockSpec((d, 128), lambda i: (0, 0)),
            pl.BlockSpec((1, 128), lambda i: (0, 0)),
        ],
        out_specs=pl.BlockSpec((NBQ, 128), lambda i: (i, 0)),
        out_shape=jax.ShapeDtypeStruct((n, 128), jnp.float32),
    )(x, wq_eff, c0)


def _moe_update(h, s_ref, wr1_ref, br_ref, we1_ref, be1_ref, we2_ref, be2_ref,
                wk1_ref, bk1_ref, wk2_ref, bk2_ref):
    s = s_ref[0] + s_ref[1]
    deg = jnp.maximum(s[:, N_EXP:N_EXP + 1], 1.0)
    logits = (
        jnp.dot(h, wr1_ref[...], preferred_element_type=jnp.float32)
        + s[:, :N_EXP] / deg
        + br_ref[...]
    )

    iota = jax.lax.broadcasted_iota(jnp.int32, (NB, N_EXP), 1)
    m1 = jnp.max(logits, axis=1, keepdims=True)
    i1 = jnp.min(jnp.where(logits == m1, iota, N_EXP), axis=1, keepdims=True)
    oh1 = iota == i1
    masked = jnp.where(oh1, -3e38, logits)
    m2 = jnp.max(masked, axis=1, keepdims=True)
    i2 = jnp.min(jnp.where(masked == m2, iota, N_EXP), axis=1, keepdims=True)
    oh2 = iota == i2
    t = jnp.exp(m2 - m1)
    g1 = 1.0 / (1.0 + t)
    g2 = 1.0 - g1
    comb = g1 * oh1.astype(jnp.float32) + g2 * oh2.astype(jnp.float32)
    conf = 1.0 / jnp.sum(jnp.exp(logits - m1), axis=1, keepdims=True)

    weak = (
        jnp.dot(
            jnp.maximum(
                jnp.dot(h, wk1_ref[...], preferred_element_type=jnp.float32)
                + bk1_ref[...],
                0.0,
            ),
            wk2_ref[...],
            preferred_element_type=jnp.float32,
        )
        + bk2_ref[...]
    )

    moe = weak
    h16 = h.astype(jnp.bfloat16)
    for e in range(N_EXP):
        hid = jnp.maximum(
            jnp.dot(h16, we1_ref[e].astype(jnp.bfloat16),
                    preferred_element_type=jnp.float32) + be1_ref[e],
            0.0,
        )
        eo = jnp.dot(hid.astype(jnp.bfloat16), we2_ref[e].astype(jnp.bfloat16),
                     preferred_element_type=jnp.float32) + be2_ref[e]
        moe = moe + comb[:, e:e + 1] * eo

    return h + conf * moe


def _layer0_body(x_ref, s_ref, win_ref, binp_ref, wr1_ref, br_ref, we1_ref,
                 be1_ref, we2_ref, be2_ref, wk1_ref, bk1_ref, wk2_ref, bk2_ref,
                 wq_ref, h_out_ref, q_ref):
    h = (jnp.dot(x_ref[...], win_ref[...], preferred_element_type=jnp.float32)
         + binp_ref[...])
    hn = _moe_update(h, s_ref, wr1_ref, br_ref, we1_ref, be1_ref, we2_ref,
                     be2_ref, wk1_ref, bk1_ref, wk2_ref, bk2_ref)
    h_out_ref[...] = hn
    ones_col = (jax.lax.broadcasted_iota(jnp.int32, (NB, 128), 1) == N_EXP
                ).astype(jnp.float32)
    q_ref[...] = jnp.dot(hn, wq_ref[...], preferred_element_type=jnp.float32) + ones_col


def _layer_last_body(h_ref, s_ref, wr1_ref, br_ref, we1_ref, be1_ref,
                     we2_ref, be2_ref, wk1_ref, bk1_ref, wk2_ref, bk2_ref,
                     wo_ref, bo_ref, o_ref):
    hn = _moe_update(h_ref[...], s_ref, wr1_ref, br_ref,
                     we1_ref, be1_ref, we2_ref, be2_ref,
                     wk1_ref, bk1_ref, wk2_ref, bk2_ref)
    o_ref[...] = jnp.dot(hn, wo_ref[...], preferred_element_type=jnp.float32) + bo_ref[...]


def _layer_specs(pre, extra):
    full = lambda *s: pl.BlockSpec(s, lambda i, _ns=len(s): (0,) * _ns)
    return [
        pl.BlockSpec((NB, HID), lambda i: (i, 0)),
        pl.BlockSpec((2, NB, QW), lambda i: (0, i, 0)),
    ] + [full(*s) for s in pre] + [full(*s) for s in (
        (HID, N_EXP),
        (1, N_EXP),
        (N_EXP, HID, HID),
        (N_EXP, 1, HID),
        (N_EXP, HID, HID),
        (N_EXP, 1, HID),
        (HID, WEAK),
        (1, WEAK),
        (WEAK, HID),
        (1, HID),
    )] + [full(*s) for s in extra]


def _layer_args(wr, br, we1, be1, we2, be2, wk1, bk1, wk2, bk2):
    return (
        wr[:HID], br.reshape(1, N_EXP),
        we1, be1.reshape(N_EXP, 1, HID), we2, be2.reshape(N_EXP, 1, HID),
        wk1, bk1.reshape(1, WEAK), wk2, bk2.reshape(1, HID),
    )


def kernel(x, edge_index, Win, bin_, Wr, br, We1, be1, We2, be2, Wk1, bk1, Wk2, bk2, Wout, bout):
    n = x.shape[0]
    src = edge_index[0]
    dst = edge_index[1]

    npad_e = EPAD - src.shape[0]
    pad_ar = jnp.arange(npad_e, dtype=jnp.int32)
    edges2d = jnp.stack([
        jnp.concatenate([src * 8, (pad_ar % n) * 8]).reshape(EPAD // CHUNK, CHUNK),
        jnp.concatenate([dst, n + pad_ar % (NPAD - n)]).reshape(EPAD // CHUNK, CHUNK),
    ])
    zeros = jnp.zeros((NPAD, QW), jnp.float32)

    wq = [jnp.pad(Wr[l][HID:], ((0, 0), (0, 128 - N_EXP))) for l in range(Wr.shape[0])]
    wq0_eff = Win @ wq[0]
    c0 = (bin_ @ wq[0] + (jnp.arange(128) == N_EXP)).reshape(1, 128).astype(jnp.float32)

    q = _q0_proj(x, wq0_eff, c0)
    s = _sc_agg(q.reshape(n * 8, QW), edges2d, zeros)

    h, q = pl.pallas_call(
        _layer0_body,
        grid=(n // NB,),
        in_specs=_layer_specs([(HID, HID), (1, HID)], [(HID, 128)]),
        out_specs=[
            pl.BlockSpec((NB, HID), lambda i: (i, 0)),
            pl.BlockSpec((NB, 128), lambda i: (i, 0)),
        ],
        out_shape=[
            jax.ShapeDtypeStruct((n, HID), jnp.float32),
            jax.ShapeDtypeStruct((n, 128), jnp.float32),
        ],
    )(x, s, Win, bin_.reshape(1, HID),
      *_layer_args(Wr[0], br[0], We1[0], be1[0], We2[0], be2[0],
                   Wk1[0], bk1[0], Wk2[0], bk2[0]), wq[1])

    s = _sc_agg(q.reshape(n * 8, QW), edges2d, zeros)

    out = pl.pallas_call(
        _layer_last_body,
        grid=(n // NB,),
        in_specs=_layer_specs([], [(HID, HID), (1, HID)]),
        out_specs=pl.BlockSpec((NB, HID), lambda i: (i, 0)),
        out_shape=jax.ShapeDtypeStruct((n, HID), jnp.float32),
    )(h, s, *_layer_args(Wr[1], br[1], We1[1], be1[1], We2[1], be2[1],
                         Wk1[1], bk1[1], Wk2[1], bk2[1]), Wout, bout.reshape(1, HID))

    return out
